# Initial kernel scaffold; baseline (speedup 1.0000x reference)
import jax
import jax.numpy as jnp
from jax.experimental import pallas as pl


def kernel(x, edge_index, W1, b1, W2, b2):
    raise NotImplementedError("write your pallas kernel here")


# speedup vs baseline: 17.1736x; 17.1736x over previous
import functools

import jax
import jax.numpy as jnp
from jax import lax
from jax.experimental import pallas as pl
from jax.experimental.pallas import tpu as pltpu
from jax.experimental.pallas import tpu_sc as plsc

_N = 10000
_D = 128
_E = 320000

_NC = 2
_NS = 16
_NW = _NC * _NS

_CHUNK = 128
_EPAD = 327680
_EPT = _EPAD // _NW
_EPC = _EPT // _CHUNK

_NPAD = 10240
_SLAB = _NPAD // _NS

@functools.cache
def _sc_mesh():
    return plsc.VectorSubcoreMesh(core_axis_name="c", subcore_axis_name="s",
                                  num_cores=_NC, num_subcores=_NS)


def _fill_f32(ref, n16, value):
    v = jnp.full((16,), value, jnp.float32)

    def body(i, carry):
        ref[pl.ds(i * 16, 16)] = v
        return carry

    lax.fori_loop(0, n16, body, 0)


def _deg_body(dst_hbm, out_hbm, idx_v, ones_v, buf_v, deg_sh):
    c = lax.axis_index("c")
    s = lax.axis_index("s")
    wid = c * _NS + s

    _fill_f32(ones_v, _CHUNK // 16, 1.0)
    _fill_f32(buf_v, _SLAB // 16, 0.0)
    pltpu.sync_copy(buf_v, deg_sh.at[pl.ds(s * _SLAB, _SLAB)])
    plsc.subcore_barrier()

    pltpu.sync_copy(dst_hbm.at[pl.ds(wid * _EPC, _EPC)], idx_v)

    def body(j, carry):
        pltpu.sync_copy(ones_v, deg_sh.at[idx_v.at[j]], add=True)
        return carry

    lax.fori_loop(0, _EPC, body, 0)
    plsc.subcore_barrier()

    pltpu.sync_copy(deg_sh.at[pl.ds(s * _SLAB, _SLAB)], buf_v)
    pltpu.sync_copy(buf_v, out_hbm.at[c, pl.ds(s * _SLAB, _SLAB)])


@functools.cache
def _deg_call():
    return pl.kernel(
        _deg_body,
        out_type=jax.ShapeDtypeStruct((_NC, _NPAD), jnp.float32),
        mesh=_sc_mesh(),
        scratch_types=[
            pltpu.VMEM((_EPC, _CHUNK), jnp.int32),
            pltpu.VMEM((_CHUNK,), jnp.float32),
            pltpu.VMEM((_SLAB,), jnp.float32),
            pltpu.VMEM_SHARED((_NPAD,), jnp.float32),
        ],
    )


_DH = _D // 2
_EPT2 = _EPAD // _NS
_EPC2 = _EPT2 // _CHUNK


def _agg_body(h_hbm, src_hbm, dst_hbm, out_hbm, src_v, dst_v, rows_v, zbuf_v,
              acc_sh, gsem):
    c = lax.axis_index("c")
    s = lax.axis_index("s")

    z16 = jnp.zeros((16,), jnp.float32)

    def zbody(i, carry):
        zbuf_v[i // (_DH // 16), pl.ds((i % (_DH // 16)) * 16, 16)] = z16
        return carry

    lax.fori_loop(0, 128 * (_DH // 16), zbody, 0)
    for k in range(_SLAB // 128):
        pltpu.sync_copy(zbuf_v, acc_sh.at[pl.ds(s * _SLAB + k * 128, 128)])
    plsc.subcore_barrier()

    pltpu.sync_copy(src_hbm.at[pl.ds(s * _EPC2, _EPC2)], src_v)
    pltpu.sync_copy(dst_hbm.at[pl.ds(s * _EPC2, _EPC2)], dst_v)

    hc = h_hbm.at[c]

    def body(j, carry):
        pltpu.async_copy(hc.at[src_v.at[j]], rows_v.at[0], gsem).wait()
        pltpu.sync_copy(rows_v.at[0], acc_sh.at[dst_v.at[j]], add=True)
        return carry

    lax.fori_loop(0, _EPC2, body, 0)
    plsc.subcore_barrier()

    base = s * _SLAB
    for k in range(_SLAB // 128):
        pltpu.sync_copy(acc_sh.at[pl.ds(base + k * 128, 128)], zbuf_v)
        pltpu.sync_copy(zbuf_v, out_hbm.at[c, pl.ds(base + k * 128, 128)])


@functools.cache
def _agg_call():
    return pl.kernel(
        _agg_body,
        out_type=jax.ShapeDtypeStruct((_NC, _NPAD, _DH), jnp.float32),
        mesh=_sc_mesh(),
        scratch_types=[
            pltpu.VMEM((_EPC2, _CHUNK), jnp.int32),
            pltpu.VMEM((_EPC2, _CHUNK), jnp.int32),
            pltpu.VMEM((2, _CHUNK, _DH), jnp.float32),
            pltpu.VMEM((128, _DH), jnp.float32),
            pltpu.VMEM_SHARED((_NPAD, _DH), jnp.float32),
            pltpu.SemaphoreType.DMA,
        ],
        compiler_params=pltpu.CompilerParams(use_tc_tiling_on_sc=False),
    )


def _tc1_body(x_ref, w_ref, degsum_ref, h_ref):
    dinv = lax.rsqrt(1.0 + degsum_ref[...])
    h = jnp.dot(x_ref[...], w_ref[...], preferred_element_type=jnp.float32)
    h_ref[...] = h * dinv


def _acc_full(acc_ref):
    return jnp.concatenate([acc_ref[0, :_N], acc_ref[1, :_N]], axis=1)


def _tc2_body(acc_ref, h1_ref, degsum_ref, b1_ref, w2_ref, out_ref):
    dinv = lax.rsqrt(1.0 + degsum_ref[...])
    z = dinv * (_acc_full(acc_ref) + h1_ref[...]) + b1_ref[...]
    z = jnp.maximum(z, 0.0)
    out_ref[...] = dinv * jnp.dot(z, w2_ref[...],
                                  preferred_element_type=jnp.float32)


def _tc3_body(acc_ref, h2_ref, degsum_ref, b2_ref, out_ref):
    dinv = lax.rsqrt(1.0 + degsum_ref[...])
    z = dinv * (_acc_full(acc_ref) + h2_ref[...]) + b2_ref[...]
    m = jnp.max(z, axis=1, keepdims=True)
    e = jnp.exp(z - m)
    lse = jnp.log(jnp.sum(e, axis=1, keepdims=True)) + m
    out_ref[...] = z - lse


def _tc_call(body, num_in, out_shape, interpret=False):
    return pl.pallas_call(
        body,
        out_shape=jax.ShapeDtypeStruct(out_shape, jnp.float32),
        in_specs=[pl.BlockSpec(memory_space=pltpu.VMEM)
                  for _ in range(num_in)],
        out_specs=pl.BlockSpec(memory_space=pltpu.VMEM),
        interpret=interpret,
    )


_tc1 = _tc_call(_tc1_body, 3, (_N, _D))
_tc2 = _tc_call(_tc2_body, 5, (_N, _D))
_tc3 = _tc_call(_tc3_body, 4, (_N, _D))


def kernel(x, edge_index, W1, b1, W2, b2):
    src = edge_index[0].astype(jnp.int32)
    dst = edge_index[1].astype(jnp.int32)

    pad = _EPAD - _E
    pad_idx = jnp.arange(pad, dtype=jnp.int32)
    src_p = jnp.concatenate([src, pad_idx % _N]).reshape(_EPAD // _CHUNK,
                                                         _CHUNK)
    dst_p = jnp.concatenate([dst, _N + pad_idx % (_NPAD - _N)]).reshape(
        _EPAD // _CHUNK, _CHUNK)

    deg_p = _deg_call()(dst_p)
    degsum = (deg_p[0, :_N] + deg_p[1, :_N]).reshape(_N, 1)

    def split(h):
        return jnp.moveaxis(h.reshape(_N, 2, _DH), 1, 0)

    h1 = _tc1(x, W1, degsum)
    acc1 = _agg_call()(split(h1), src_p, dst_p)
    h2 = _tc2(acc1, h1, degsum, b1.reshape(1, _D), W2)
    acc2 = _agg_call()(split(h2), src_p, dst_p)
    return _tc3(acc2, h2, degsum, b2.reshape(1, _D))
